# baseline (speedup 1.0000x reference)
import functools

import jax
import jax.numpy as jnp
from jax import lax
from jax.experimental import pallas as pl
from jax.experimental.pallas import tpu as pltpu
from jax.experimental.pallas import tpu_sc as plsc

_INFO = plsc.get_sparse_core_info()
_NC = _INFO.num_cores
_NS = _INFO.num_subcores
_NW = _NC * _NS


def _gather_native(table, bp, b, l, v, d):
    lt_n, bt_n = l // 8, b // 128
    oct_n = d // 8
    units = l * 4
    per_w = units // _NW
    q_bt = bt_n // 4
    mesh = plsc.VectorSubcoreMesh(core_axis_name="c", subcore_axis_name="s")

    @functools.partial(
        pl.kernel,
        mesh=mesh,
        out_type=jax.ShapeDtypeStruct((l, oct_n, bt_n, 8, 128), jnp.float32),
        scratch_types=[
            pltpu.VMEM((q_bt, 128), jnp.int32),
            pltpu.VMEM((2, 1024, d), jnp.float32),
            pltpu.VMEM((8, 8, 128), jnp.float32),
            pltpu.SemaphoreType.DMA((2,)),
        ],
        compiler_params=pltpu.CompilerParams(
            use_tc_tiling_on_sc=False, needs_layout_passes=False),
    )
    def k(table_hbm, bp_hbm, out_hbm, idx_v, rows_v, oct_v, gsem):
        wid = lax.axis_index("s") * _NC + lax.axis_index("c")
        iota = lax.iota(jnp.int32, 16)

        def start_gather(kbuf, kchunk):
            for t in range(8):
                pltpu.async_copy(
                    table_hbm.at[idx_v.at[kchunk * 8 + t]],
                    rows_v.at[kbuf, pl.ds(t * 128, 128)],
                    gsem.at[kbuf])

        def wait_gather(kbuf, kchunk):
            for t in range(8):
                pltpu.make_async_copy(
                    table_hbm.at[idx_v.at[kchunk * 8 + t]],
                    rows_v.at[kbuf, pl.ds(t * 128, 128)],
                    gsem.at[kbuf]).wait()

        def unit_body(u, carry):
            uid = wid * per_w + u
            ul = uid // 4
            uq = uid - ul * 4
            lt = ul // 8
            li = ul - lt * 8
            bt0 = uq * q_bt
            pltpu.sync_copy(bp_hbm.at[lt, pl.ds(bt0, q_bt), li, :], idx_v)
            start_gather(0, 0)

            for kc in range(4):
                kbuf = kc % 2
                if kc + 1 < 4:
                    start_gather(1 - kbuf, kc + 1)
                wait_gather(kbuf, kc)
                for c_oct in range(oct_n):
                    def bt_body(bt, carry3, kbuf=kbuf, c_oct=c_oct):
                        def cl_body(cl, carry4):
                            c = c_oct * 8 + cl
                            for g in range(8):
                                rvec = bt * 128 + g * 16 + iota
                                cvec = jnp.full((16,), c, jnp.int32)
                                vals = plsc.load_gather(
                                    rows_v.at[kbuf], [rvec, cvec])
                                oct_v[bt, cl, pl.ds(g * 16, 16)] = vals
                            return carry4
                        lax.fori_loop(0, 8, cl_body, 0)
                        return carry3
                    lax.fori_loop(0, 8, bt_body, 0)
                    pltpu.sync_copy(
                        oct_v,
                        out_hbm.at[ul, c_oct, pl.ds(bt0 + kc * 8, 8), :, :])
            return carry

        lax.fori_loop(0, per_w, unit_body, 0)

    return k(table, bp)


def kernel(batch, lengths, table):
    b, l = batch.shape
    v, d = table.shape
    bp = batch.reshape(b // 128, 128, l // 8, 8).transpose(2, 0, 3, 1)
    op = _gather_native(table, bp, b, l, v, d)
    return op.transpose(2, 4, 0, 1, 3).reshape(b, l, d)


# speedup vs baseline: 1.0589x; 1.0589x over previous
import functools

import jax
import jax.numpy as jnp
from jax import lax
from jax.experimental import pallas as pl
from jax.experimental.pallas import tpu as pltpu
from jax.experimental.pallas import tpu_sc as plsc

_INFO = plsc.get_sparse_core_info()
_NC = _INFO.num_cores
_NS = _INFO.num_subcores
_NW = _NC * _NS


def _gather_native(table, bp, b, l, v, d):
    lt_n, bt_n = l // 8, b // 128
    oct_n = d // 8
    units = l * 4
    per_w = units // _NW
    q_bt = bt_n // 4
    mesh = plsc.VectorSubcoreMesh(core_axis_name="c", subcore_axis_name="s")

    n_ch = 8
    ch_bt = q_bt // n_ch
    ch_rows = ch_bt * 128

    @functools.partial(
        pl.kernel,
        mesh=mesh,
        out_type=jax.ShapeDtypeStruct((l, oct_n, bt_n, 8, 128), jnp.float32),
        scratch_types=[
            pltpu.VMEM((q_bt, 128), jnp.int32),
            pltpu.VMEM((2, ch_rows, d), jnp.float32),
            pltpu.VMEM((2, oct_n, ch_bt, 8, 128), jnp.float32),
            pltpu.SemaphoreType.DMA((2,)),
            pltpu.SemaphoreType.DMA((2,)),
        ],
        compiler_params=pltpu.CompilerParams(
            use_tc_tiling_on_sc=False, needs_layout_passes=False),
    )
    def k(table_hbm, bp_hbm, out_hbm, idx_v, rows_v, trans_v, gsem, wsem):
        wid = lax.axis_index("s") * _NC + lax.axis_index("c")
        iota = lax.iota(jnp.int32, 16)

        def start_gather(buf, kchunk):
            for t in range(ch_bt):
                pltpu.async_copy(
                    table_hbm.at[idx_v.at[kchunk * ch_bt + t]],
                    rows_v.at[buf, pl.ds(t * 128, 128)],
                    gsem.at[buf])

        def wait_gather(buf, kchunk):
            for t in range(ch_bt):
                pltpu.make_async_copy(
                    table_hbm.at[idx_v.at[kchunk * ch_bt + t]],
                    rows_v.at[buf, pl.ds(t * 128, 128)],
                    gsem.at[buf]).wait()

        def start_writes(buf, ul, c0, kc):
            for co in range(oct_n):
                pltpu.async_copy(
                    trans_v.at[buf, co],
                    out_hbm.at[ul, co, pl.ds(c0 + kc * ch_bt, ch_bt), :, :],
                    wsem.at[buf])

        def wait_writes(buf, ul, c0, kc):
            for co in range(oct_n):
                pltpu.make_async_copy(
                    trans_v.at[buf, co],
                    out_hbm.at[ul, co, pl.ds(c0 + kc * ch_bt, ch_bt), :, :],
                    wsem.at[buf]).wait()

        def transpose_chunk(buf):
            rows = rows_v.at[buf]

            def c_body(c, carry):
                co = c // 8
                cl = c - co * 8
                cvec = jnp.full((16,), c, jnp.int32)
                for bt in range(ch_bt):
                    for g in range(8):
                        rvec = (bt * 128 + g * 16) + iota
                        vals = plsc.load_gather(rows, [rvec, cvec])
                        trans_v[buf, co, bt, cl, pl.ds(g * 16, 16)] = vals
                return carry

            lax.fori_loop(0, d, c_body, 0)

        def unit_body(u, carry):
            uid = wid * per_w + u
            ul = uid // 4
            uq = uid - ul * 4
            lt = ul // 8
            li = ul - lt * 8
            bt0 = uq * q_bt
            pltpu.sync_copy(bp_hbm.at[lt, pl.ds(bt0, q_bt), li, :], idx_v)
            start_gather(0, 0)

            for kc in range(n_ch):
                buf = kc % 2
                if kc + 1 < n_ch:
                    start_gather(1 - buf, kc + 1)
                wait_gather(buf, kc)
                if kc >= 2:
                    wait_writes(buf, ul, bt0, kc - 2)
                transpose_chunk(buf)
                start_writes(buf, ul, bt0, kc)
            wait_writes(0, ul, bt0, n_ch - 2)
            wait_writes(1, ul, bt0, n_ch - 1)
            return carry

        lax.fori_loop(0, per_w, unit_body, 0)

    return k(table, bp)


def kernel(batch, lengths, table):
    b, l = batch.shape
    v, d = table.shape
    bp = batch.reshape(b // 128, 128, l // 8, 8).transpose(2, 0, 3, 1)
    op = _gather_native(table, bp, b, l, v, d)
    return op.transpose(2, 4, 0, 1, 3).reshape(b, l, d)


# speedup vs baseline: 2.3055x; 2.1773x over previous
import functools

import jax
import jax.numpy as jnp
from jax import lax
from jax.experimental import pallas as pl
from jax.experimental.pallas import tpu as pltpu
from jax.experimental.pallas import tpu_sc as plsc

_INFO = plsc.get_sparse_core_info()
_NC = _INFO.num_cores
_NS = _INFO.num_subcores
_NW = _NC * _NS


def _gather_native(table, bp, b, l, v, d):
    lt_n, bt_n = l // 8, b // 128
    oct_n = d // 8
    units = l * 4
    per_w = units // _NW
    q_bt = bt_n // 4
    mesh = plsc.VectorSubcoreMesh(core_axis_name="c", subcore_axis_name="s")

    n_ch = 8
    ch_bt = q_bt // n_ch
    ch_rows = ch_bt * 128

    t_rows = ch_bt * d

    @functools.partial(
        pl.kernel,
        mesh=mesh,
        out_type=jax.ShapeDtypeStruct((l, oct_n, bt_n * 8, 128), jnp.float32),
        scratch_types=[
            pltpu.VMEM((q_bt, 128), jnp.int32),
            pltpu.VMEM((2, ch_rows, d), jnp.float32),
            pltpu.VMEM((2, t_rows, 129), jnp.float32),
            pltpu.SemaphoreType.DMA((2,)),
            pltpu.SemaphoreType.DMA((2,)),
        ],
        compiler_params=pltpu.CompilerParams(
            use_tc_tiling_on_sc=False, needs_layout_passes=False),
    )
    def k(table_hbm, bp_hbm, out_hbm, idx_v, rows_v, trans_v, gsem, wsem):
        wid = lax.axis_index("s") * _NC + lax.axis_index("c")
        iota = lax.iota(jnp.int32, 16)

        def start_gather(buf, kchunk):
            for t in range(ch_bt):
                pltpu.async_copy(
                    table_hbm.at[idx_v.at[kchunk * ch_bt + t]],
                    rows_v.at[buf, pl.ds(t * 128, 128)],
                    gsem.at[buf])

        def wait_gather(buf, kchunk):
            for t in range(ch_bt):
                pltpu.make_async_copy(
                    table_hbm.at[idx_v.at[kchunk * ch_bt + t]],
                    rows_v.at[buf, pl.ds(t * 128, 128)],
                    gsem.at[buf]).wait()

        def _wr(copyfn, buf, ul, c0, kc):
            for co in range(oct_n):
                for bt in range(ch_bt):
                    copyfn(
                        trans_v.at[buf, pl.ds(bt * d + co * 8, 8),
                                   pl.ds(0, 128)],
                        out_hbm.at[ul, co,
                                   pl.ds((c0 + kc * ch_bt + bt) * 8, 8), :],
                        wsem.at[buf])

        def start_writes(buf, ul, c0, kc):
            _wr(pltpu.async_copy, buf, ul, c0, kc)

        def wait_writes(buf, ul, c0, kc):
            _wr(lambda s, t, m: pltpu.make_async_copy(s, t, m).wait(),
                buf, ul, c0, kc)

        def transpose_chunk(buf):
            trans2 = trans_v.at[buf]
            for bt in range(ch_bt):
                qv0 = iota + (bt * d)
                qv1 = iota + (bt * d + 16)

                def g_body(g, bv, bt=bt, qv0=qv0, qv1=qv1):
                    for j in range(8):
                        r = bt * 128 + g * 8 + j
                        v0 = rows_v[buf, r, pl.ds(0, 16)]
                        plsc.store_scatter(trans2, [qv0, bv], v0)
                        v1 = rows_v[buf, r, pl.ds(16, 16)]
                        plsc.store_scatter(trans2, [qv1, bv], v1)
                        bv = bv + 1
                    return bv

                lax.fori_loop(0, 16, g_body, jnp.zeros((16,), jnp.int32))

        def unit_body(u, carry):
            uid = wid * per_w + u
            ul = uid // 4
            uq = uid - ul * 4
            lt = ul // 8
            li = ul - lt * 8
            bt0 = uq * q_bt
            pltpu.sync_copy(bp_hbm.at[lt, pl.ds(bt0, q_bt), li, :], idx_v)
            start_gather(0, 0)

            for kc in range(n_ch):
                buf = kc % 2
                if kc + 1 < n_ch:
                    start_gather(1 - buf, kc + 1)
                wait_gather(buf, kc)
                if kc >= 2:
                    wait_writes(buf, ul, bt0, kc - 2)
                transpose_chunk(buf)
                start_writes(buf, ul, bt0, kc)
            wait_writes(0, ul, bt0, n_ch - 2)
            wait_writes(1, ul, bt0, n_ch - 1)
            return carry

        lax.fori_loop(0, per_w, unit_body, 0)

    return k(table, bp)


def kernel(batch, lengths, table):
    b, l = batch.shape
    v, d = table.shape
    bp = batch.reshape(b // 128, 128, l // 8, 8).transpose(2, 0, 3, 1)
    op = _gather_native(table, bp, b, l, v, d)
    op = op.reshape(l, d // 8, b // 128, 8, 128)
    return op.transpose(2, 4, 0, 1, 3).reshape(b, l, d)


# speedup vs baseline: 3.4076x; 1.4781x over previous
import functools

import jax
import jax.numpy as jnp
from jax import lax
from jax.experimental import pallas as pl
from jax.experimental.pallas import tpu as pltpu
from jax.experimental.pallas import tpu_sc as plsc

_INFO = plsc.get_sparse_core_info()
_NC = _INFO.num_cores
_NS = _INFO.num_subcores
_NW = _NC * _NS


def _gather_native(table, bp, b, l, v, d):
    lt_n, bt_n = l // 8, b // 128
    oct_n = d // 8
    units = l * 4
    per_w = units // _NW
    q_bt = bt_n // 4
    mesh = plsc.VectorSubcoreMesh(core_axis_name="c", subcore_axis_name="s")

    n_ch = 8
    ch_bt = q_bt // n_ch
    ch_rows = ch_bt * 128

    t_rows = ch_bt * d

    @functools.partial(
        pl.kernel,
        mesh=mesh,
        out_type=jax.ShapeDtypeStruct((l, oct_n, bt_n * 8, 128), jnp.float32),
        scratch_types=[
            pltpu.VMEM((q_bt, 128), jnp.int32),
            pltpu.VMEM((2, ch_rows, d), jnp.float32),
            pltpu.VMEM((2, t_rows, 129), jnp.float32),
            pltpu.SemaphoreType.DMA((2,)),
            pltpu.SemaphoreType.DMA((2,)),
        ],
        compiler_params=pltpu.CompilerParams(
            use_tc_tiling_on_sc=False, needs_layout_passes=False),
    )
    def k(table_hbm, bp_hbm, out_hbm, idx_v, rows_v, trans_v, gsem, wsem):
        wid = lax.axis_index("s") * _NC + lax.axis_index("c")
        iota = lax.iota(jnp.int32, 16)

        def start_gather(buf, kchunk):
            for t in range(ch_bt):
                pltpu.async_copy(
                    table_hbm.at[idx_v.at[kchunk * ch_bt + t]],
                    rows_v.at[buf, pl.ds(t * 128, 128)],
                    gsem.at[buf])

        def wait_gather(buf, kchunk):
            for t in range(ch_bt):
                pltpu.make_async_copy(
                    table_hbm.at[idx_v.at[kchunk * ch_bt + t]],
                    rows_v.at[buf, pl.ds(t * 128, 128)],
                    gsem.at[buf]).wait()

        def _wr(copyfn, buf, ul, c0, kc):
            for co in range(oct_n):
                for bt in range(ch_bt):
                    copyfn(
                        trans_v.at[buf, pl.ds(bt * d + co * 8, 8),
                                   pl.ds(0, 128)],
                        out_hbm.at[ul, co,
                                   pl.ds((c0 + kc * ch_bt + bt) * 8, 8), :],
                        wsem.at[buf])

        def start_writes(buf, ul, c0, kc):
            _wr(pltpu.async_copy, buf, ul, c0, kc)

        def wait_writes(buf, ul, c0, kc):
            _wr(lambda s, t, m: pltpu.make_async_copy(s, t, m).wait(),
                buf, ul, c0, kc)

        def transpose_chunk(buf):
            trans2 = trans_v.at[buf]
            for bt in range(ch_bt):
                qv0 = iota + (bt * d)
                qv1 = iota + (bt * d + 16)

                def g_body(g, bv, bt=bt, qv0=qv0, qv1=qv1):
                    staged = []
                    for j in range(8):
                        r = bt * 128 + g * 8 + j
                        staged.append((rows_v[buf, r, pl.ds(0, 16)],
                                       rows_v[buf, r, pl.ds(16, 16)], bv))
                        bv = bv + 1
                    for v0, v1, bvj in staged:
                        plsc.store_scatter(trans2, [qv0, bvj], v0)
                        plsc.store_scatter(trans2, [qv1, bvj], v1)
                    return bv

                lax.fori_loop(0, 16, g_body, jnp.zeros((16,), jnp.int32))

        def unit_body(u, carry):
            uid = wid * per_w + u
            ul = uid // 4
            uq = uid - ul * 4
            lt = ul // 8
            li = ul - lt * 8
            bt0 = uq * q_bt
            pltpu.sync_copy(bp_hbm.at[lt, pl.ds(bt0, q_bt), li, :], idx_v)
            start_gather(0, 0)

            for kc in range(n_ch):
                buf = kc % 2
                if kc + 1 < n_ch:
                    start_gather(1 - buf, kc + 1)
                wait_gather(buf, kc)
                if kc >= 2:
                    wait_writes(buf, ul, bt0, kc - 2)
                transpose_chunk(buf)
                start_writes(buf, ul, bt0, kc)
            wait_writes(0, ul, bt0, n_ch - 2)
            wait_writes(1, ul, bt0, n_ch - 1)
            return carry

        lax.fori_loop(0, per_w, unit_body, 0)

    return k(table, bp)


def kernel(batch, lengths, table):
    b, l = batch.shape
    v, d = table.shape
    bp = batch.reshape(b // 128, 128, l // 8, 8).transpose(2, 0, 3, 1)
    op = _gather_native(table, bp, b, l, v, d)
    op = op.reshape(l, d // 8, b // 128, 8, 128)
    return op.transpose(2, 4, 0, 1, 3).reshape(b, l, d)


# speedup vs baseline: 3.4121x; 1.0013x over previous
import functools

import jax
import jax.numpy as jnp
from jax import lax
from jax.experimental import pallas as pl
from jax.experimental.pallas import tpu as pltpu
from jax.experimental.pallas import tpu_sc as plsc

_INFO = plsc.get_sparse_core_info()
_NC = _INFO.num_cores
_NS = _INFO.num_subcores
_NW = _NC * _NS


def _gather_native(table, bp, b, l, v, d):
    lt_n, bt_n = l // 8, b // 128
    oct_n = d // 8
    units = l * 4
    per_w = units // _NW
    q_bt = bt_n // 4
    mesh = plsc.VectorSubcoreMesh(core_axis_name="c", subcore_axis_name="s")

    n_ch = 8
    ch_bt = q_bt // n_ch
    ch_rows = ch_bt * 128

    t_rows = ch_bt * d

    @functools.partial(
        pl.kernel,
        mesh=mesh,
        out_type=jax.ShapeDtypeStruct((l, oct_n, bt_n * 8, 128), jnp.float32),
        scratch_types=[
            pltpu.VMEM((q_bt, 128), jnp.int32),
            pltpu.VMEM((2, ch_bt, 128, d), jnp.float32),
            pltpu.VMEM((2, t_rows, 129), jnp.float32),
            pltpu.SemaphoreType.DMA((2,)),
            pltpu.SemaphoreType.DMA((2,)),
        ],
        compiler_params=pltpu.CompilerParams(
            use_tc_tiling_on_sc=False, needs_layout_passes=False),
    )
    def k(table_hbm, bp_hbm, out_hbm, idx_v, rows_v, trans_v, gsem, wsem):
        wid = lax.axis_index("s") * _NC + lax.axis_index("c")
        iota = lax.iota(jnp.int32, 16)

        def start_gather(buf, kchunk):
            for t in range(ch_bt):
                pltpu.async_copy(
                    table_hbm.at[idx_v.at[kchunk * ch_bt + t]],
                    rows_v.at[buf, t],
                    gsem.at[buf])

        def wait_gather(buf, kchunk):
            for t in range(ch_bt):
                pltpu.make_async_copy(
                    table_hbm.at[idx_v.at[kchunk * ch_bt + t]],
                    rows_v.at[buf, t],
                    gsem.at[buf]).wait()

        def _wr(copyfn, buf, ul, c0, kc):
            for co in range(oct_n):
                for bt in range(ch_bt):
                    copyfn(
                        trans_v.at[buf, pl.ds(bt * d + co * 8, 8),
                                   pl.ds(0, 128)],
                        out_hbm.at[ul, co,
                                   pl.ds((c0 + kc * ch_bt + bt) * 8, 8), :],
                        wsem.at[buf])

        def start_writes(buf, ul, c0, kc):
            _wr(pltpu.async_copy, buf, ul, c0, kc)

        def wait_writes(buf, ul, c0, kc):
            _wr(lambda s, t, m: pltpu.make_async_copy(s, t, m).wait(),
                buf, ul, c0, kc)

        def transpose_chunk(buf):
            trans2 = trans_v.at[buf]
            for bt in range(ch_bt):
                qv0 = iota + (bt * d)
                qv1 = iota + (bt * d + 16)

                def g_body(g, bv, bt=bt, qv0=qv0, qv1=qv1):
                    staged = []
                    for j in range(8):
                        r = g * 8 + j
                        staged.append((rows_v[buf, bt, r, pl.ds(0, 16)],
                                       rows_v[buf, bt, r, pl.ds(16, 16)], bv))
                        bv = bv + 1
                    for v0, v1, bvj in staged:
                        plsc.store_scatter(trans2, [qv0, bvj], v0)
                        plsc.store_scatter(trans2, [qv1, bvj], v1)
                    return bv

                lax.fori_loop(0, 16, g_body, jnp.zeros((16,), jnp.int32))

        def unit_body(u, carry):
            uid = wid * per_w + u
            ul = uid // 4
            uq = uid - ul * 4
            lt = ul // 8
            li = ul - lt * 8
            bt0 = uq * q_bt
            pltpu.sync_copy(bp_hbm.at[lt, pl.ds(bt0, q_bt), li, :], idx_v)
            start_gather(0, 0)

            for kc in range(n_ch):
                buf = kc % 2
                if kc + 1 < n_ch:
                    start_gather(1 - buf, kc + 1)
                wait_gather(buf, kc)
                if kc >= 2:
                    wait_writes(buf, ul, bt0, kc - 2)
                transpose_chunk(buf)
                start_writes(buf, ul, bt0, kc)
            wait_writes(0, ul, bt0, n_ch - 2)
            wait_writes(1, ul, bt0, n_ch - 1)
            return carry

        lax.fori_loop(0, per_w, unit_body, 0)

    return k(table, bp)


def kernel(batch, lengths, table):
    b, l = batch.shape
    v, d = table.shape
    bp = batch.reshape(b // 128, 128, l // 8, 8).transpose(2, 0, 3, 1)
    op = _gather_native(table, bp, b, l, v, d)
    op = op.reshape(l, d // 8, b // 128, 8, 128)
    return op.transpose(2, 4, 0, 1, 3).reshape(b, l, d)
